# Initial kernel scaffold; baseline (speedup 1.0000x reference)
import jax
import jax.numpy as jnp
from jax.experimental import pallas as pl


def kernel(x, in_proj_w, conv_w, conv_b, x_proj_w, dt_proj_w, dt_proj_b, A_log, D, out_proj_w):
    raise NotImplementedError("write your pallas kernel here")


# speedup vs baseline: 9.4187x; 9.4187x over previous
import jax
import jax.numpy as jnp
from jax.experimental import pallas as pl
from jax.experimental.pallas import tpu as pltpu

_DM = 1024
_DI = 2048
_DS = 16
_DC = 4
_DR = 64
_B = 2
_L = 512
_TCH = 8
_NCH = _L // _TCH
_DBLK = 512
_NJ = _DI // _DBLK
_VMEM = 52 * 1024 * 1024


def _cp(sem):
    return pltpu.CompilerParams(dimension_semantics=sem,
                                vmem_limit_bytes=_VMEM)


def _matmul_k(x_ref, w_ref, o_ref):
    o_ref[0] = jnp.dot(x_ref[0], w_ref[...],
                       preferred_element_type=jnp.float32)


def _conv_k(u_ref, cw_ref, cb_ref, o_ref):
    u = u_ref[0]
    acc = cb_ref[...] + cw_ref[_DC - 1:_DC, :] * u
    for s in range(1, _DC):
        ush = jnp.concatenate(
            [jnp.zeros((s, u.shape[1]), jnp.float32), u[:_L - s, :]], axis=0)
        acc = acc + cw_ref[_DC - 1 - s:_DC - s, :] * ush
    o_ref[0] = acc * (1.0 / (1.0 + jnp.exp(-acc)))


def _xproj_k(u_ref, wdt_ref, wb_ref, wc_ref, wdtp_ref, bdt_ref,
             dlt_ref, bm_ref, cm_ref):
    u = u_ref[0]
    dt = jnp.dot(u, wdt_ref[...], preferred_element_type=jnp.float32)
    bm_ref[0] = jnp.dot(u, wb_ref[...], preferred_element_type=jnp.float32)
    cm_ref[0] = jnp.dot(u, wc_ref[...], preferred_element_type=jnp.float32)
    pre = jnp.dot(dt, wdtp_ref[...],
                  preferred_element_type=jnp.float32) + bdt_ref[...]
    dlt_ref[0] = jnp.maximum(pre, 0.0) + jnp.log1p(jnp.exp(-jnp.abs(pre)))


def _scan_k(dlt_ref, u_ref, z_ref, bct_ref, alog_ref, d_ref, y_ref):
    neg_a = -jnp.exp(alog_ref[...])
    dp = d_ref[...]

    def chunk(c, h):
        t0 = pl.multiple_of(c * _TCH, _TCH)
        d8 = dlt_ref[0, pl.ds(t0, _TCH), :]
        u8 = u_ref[0, pl.ds(t0, _TCH), :]
        z8 = z_ref[0, pl.ds(t0, _TCH), :]
        g8 = z8 * (1.0 / (1.0 + jnp.exp(-z8)))
        bc8 = bct_ref[0, c]
        du8 = d8 * u8
        da8 = jnp.exp(d8.reshape(_TCH, 1, _DBLK) *
                      neg_a.reshape(1, _DS, _DBLK))
        rows = []
        for r in range(_TCH):
            bcol = jnp.broadcast_to(bc8[0:_DS, r:r + 1], (_DS, _DBLK))
            ccol = jnp.broadcast_to(bc8[_DS:2 * _DS, r:r + 1], (_DS, _DBLK))
            dub = jnp.broadcast_to(du8[r:r + 1, :], (_DS, _DBLK))
            h = da8[r] * h + bcol * dub
            yr = jnp.sum(ccol * h, axis=0, keepdims=True)
            rows.append((yr + u8[r:r + 1, :] * dp) * g8[r:r + 1, :])
        y_ref[0, pl.ds(t0, _TCH), :] = jnp.concatenate(rows, axis=0)
        return h

    jax.lax.fori_loop(0, _NCH, chunk, jnp.zeros((_DS, _DBLK), jnp.float32))


def _layer(x, wi, cw, cb, wx, wdt, bdt, alog, dvec, wo):
    f32 = jnp.float32
    xz = pl.pallas_call(
        _matmul_k,
        grid=(_B, 2 * _DI // 1024),
        in_specs=[
            pl.BlockSpec((1, _L, _DM), lambda b, j: (b, 0, 0)),
            pl.BlockSpec((_DM, 1024), lambda b, j: (0, j)),
        ],
        out_specs=pl.BlockSpec((1, _L, 1024), lambda b, j: (b, 0, j)),
        out_shape=jax.ShapeDtypeStruct((_B, _L, 2 * _DI), f32),
        compiler_params=_cp(("parallel", "parallel")),
    )(x, wi.T)

    u = pl.pallas_call(
        _conv_k,
        grid=(_B, _DI // 1024),
        in_specs=[
            pl.BlockSpec((1, _L, 1024), lambda b, j: (b, 0, j)),
            pl.BlockSpec((_DC, 1024), lambda b, j: (0, j)),
            pl.BlockSpec((1, 1024), lambda b, j: (0, j)),
        ],
        out_specs=pl.BlockSpec((1, _L, 1024), lambda b, j: (b, 0, j)),
        out_shape=jax.ShapeDtypeStruct((_B, _L, _DI), f32),
        compiler_params=_cp(("parallel", "parallel")),
    )(xz, cw.T, cb.reshape(1, _DI))

    wxt = wx.T
    dlt, bm, cm = pl.pallas_call(
        _xproj_k,
        grid=(_B,),
        in_specs=[
            pl.BlockSpec((1, _L, _DI), lambda b: (b, 0, 0)),
            pl.BlockSpec((_DI, _DR), lambda b: (0, 0)),
            pl.BlockSpec((_DI, _DS), lambda b: (0, 0)),
            pl.BlockSpec((_DI, _DS), lambda b: (0, 0)),
            pl.BlockSpec((_DR, _DI), lambda b: (0, 0)),
            pl.BlockSpec((1, _DI), lambda b: (0, 0)),
        ],
        out_specs=[
            pl.BlockSpec((1, _L, _DI), lambda b: (b, 0, 0)),
            pl.BlockSpec((1, _L, _DS), lambda b: (b, 0, 0)),
            pl.BlockSpec((1, _L, _DS), lambda b: (b, 0, 0)),
        ],
        out_shape=[
            jax.ShapeDtypeStruct((_B, _L, _DI), f32),
            jax.ShapeDtypeStruct((_B, _L, _DS), f32),
            jax.ShapeDtypeStruct((_B, _L, _DS), f32),
        ],
        compiler_params=_cp(("parallel",)),
    )(u, wxt[:, :_DR], wxt[:, _DR:_DR + _DS], wxt[:, _DR + _DS:],
      wdt.T, bdt.reshape(1, _DI))

    bct = jnp.concatenate(
        [bm.reshape(_B, _NCH, _TCH, _DS).transpose(0, 1, 3, 2),
         cm.reshape(_B, _NCH, _TCH, _DS).transpose(0, 1, 3, 2)], axis=2)

    y = pl.pallas_call(
        _scan_k,
        grid=(_B, _NJ),
        in_specs=[
            pl.BlockSpec((1, _L, _DBLK), lambda b, j: (b, 0, j)),
            pl.BlockSpec((1, _L, _DBLK), lambda b, j: (b, 0, j)),
            pl.BlockSpec((1, _L, _DBLK), lambda b, j: (b, 0, _NJ + j)),
            pl.BlockSpec((1, _NCH, 2 * _DS, _TCH), lambda b, j: (b, 0, 0, 0)),
            pl.BlockSpec((_DS, _DBLK), lambda b, j: (0, j)),
            pl.BlockSpec((1, _DBLK), lambda b, j: (0, j)),
        ],
        out_specs=pl.BlockSpec((1, _L, _DBLK), lambda b, j: (b, 0, j)),
        out_shape=jax.ShapeDtypeStruct((_B, _L, _DI), f32),
        compiler_params=_cp(("parallel", "parallel")),
    )(dlt, u, xz, bct, alog.T, dvec.reshape(1, _DI))

    return pl.pallas_call(
        _matmul_k,
        grid=(_B,),
        in_specs=[
            pl.BlockSpec((1, _L, _DI), lambda b: (b, 0, 0)),
            pl.BlockSpec((_DI, _DM), lambda b: (0, 0)),
        ],
        out_specs=pl.BlockSpec((1, _L, _DM), lambda b: (b, 0, 0)),
        out_shape=jax.ShapeDtypeStruct((_B, _L, _DM), f32),
        compiler_params=_cp(("parallel",)),
    )(y, wo.T)


def kernel(x, in_proj_w, conv_w, conv_b, x_proj_w, dt_proj_w, dt_proj_b,
           A_log, D, out_proj_w):
    for i in range(4):
        x = _layer(x, in_proj_w[i], conv_w[i], conv_b[i], x_proj_w[i],
                   dt_proj_w[i], dt_proj_b[i], A_log[i], D[i], out_proj_w[i])
    return x


# speedup vs baseline: 11.7008x; 1.2423x over previous
import jax
import jax.numpy as jnp
from jax.experimental import pallas as pl
from jax.experimental.pallas import tpu as pltpu

_DM = 1024
_DI = 2048
_DS = 16
_DC = 4
_DR = 64
_B = 2
_L = 512
_TCH = 8
_NCH = _L // _TCH
_DBLK = 512
_NJ = _DI // _DBLK
_VMEM = 52 * 1024 * 1024

_DNT = (((1,), (1,)), ((), ()))


def _cp(sem):
    return pltpu.CompilerParams(dimension_semantics=sem,
                                vmem_limit_bytes=_VMEM)


def _dot_t(x, w):
    return jax.lax.dot_general(x, w, _DNT,
                               preferred_element_type=jnp.float32)


def _silu(v):
    return v * (1.0 / (1.0 + jnp.exp(-v)))


def _inproj_k(x_ref, w_ref, o_ref):
    o_ref[0] = _dot_t(x_ref[0], w_ref[...])


def _mid_k(xz_ref, cw_ref, cb_ref, wx_ref, wdt_ref, bdt_ref,
           u_ref, dlt_ref, bm_ref, cm_ref):
    u = xz_ref[0]
    acc = cb_ref[...] + cw_ref[_DC - 1:_DC, :] * u
    for s in range(1, _DC):
        ush = jnp.concatenate(
            [jnp.zeros((s, _DI), jnp.float32), u[:_L - s, :]], axis=0)
        acc = acc + cw_ref[_DC - 1 - s:_DC - s, :] * ush
    uc = _silu(acc)
    u_ref[0] = uc
    xdbl = _dot_t(uc, wx_ref[...])
    bm_ref[0] = xdbl[:, _DR:_DR + _DS]
    cm_ref[0] = xdbl[:, _DR + _DS:]
    pre = _dot_t(xdbl[:, :_DR], wdt_ref[...]) + bdt_ref[...]
    dlt_ref[0] = jnp.maximum(pre, 0.0) + jnp.log1p(jnp.exp(-jnp.abs(pre)))


def _scan_out_k(dlt_ref, u_ref, z_ref, bct_ref, alog_ref, d_ref, wo_ref,
                o_ref, y_scr):
    j = pl.program_id(1)
    neg_a = -jnp.exp(alog_ref[...])
    dp = d_ref[...]

    def chunk(c, h):
        t0 = pl.multiple_of(c * _TCH, _TCH)
        d8 = dlt_ref[0, pl.ds(t0, _TCH), :]
        u8 = u_ref[0, pl.ds(t0, _TCH), :]
        g8 = _silu(z_ref[0, pl.ds(t0, _TCH), :])
        bc8 = bct_ref[0, c]
        du8 = d8 * u8
        da8 = jnp.exp(d8.reshape(_TCH, 1, _DBLK) *
                      neg_a.reshape(1, _DS, _DBLK))
        rows = []
        for r in range(_TCH):
            bcol = jnp.broadcast_to(bc8[0:_DS, r:r + 1], (_DS, _DBLK))
            ccol = jnp.broadcast_to(bc8[_DS:2 * _DS, r:r + 1], (_DS, _DBLK))
            dub = jnp.broadcast_to(du8[r:r + 1, :], (_DS, _DBLK))
            h = da8[r] * h + bcol * dub
            yr = jnp.sum(ccol * h, axis=0, keepdims=True)
            rows.append((yr + u8[r:r + 1, :] * dp) * g8[r:r + 1, :])
        y_scr[pl.ds(t0, _TCH), :] = jnp.concatenate(rows, axis=0)
        return h

    jax.lax.fori_loop(0, _NCH, chunk, jnp.zeros((_DS, _DBLK), jnp.float32))
    contrib = _dot_t(y_scr[...], wo_ref[...])

    @pl.when(j == 0)
    def _():
        o_ref[0] = contrib

    @pl.when(j > 0)
    def _():
        o_ref[0] = o_ref[0] + contrib


def _layer(x, wi, cw, cb, wx, wdt, bdt, alog, dvec, wo):
    f32 = jnp.float32
    xz = pl.pallas_call(
        _inproj_k,
        grid=(_B, 2 * _DI // 1024),
        in_specs=[
            pl.BlockSpec((1, _L, _DM), lambda b, j: (b, 0, 0)),
            pl.BlockSpec((1024, _DM), lambda b, j: (j, 0)),
        ],
        out_specs=pl.BlockSpec((1, _L, 1024), lambda b, j: (b, 0, j)),
        out_shape=jax.ShapeDtypeStruct((_B, _L, 2 * _DI), f32),
        compiler_params=_cp(("parallel", "parallel")),
    )(x, wi)

    u, dlt, bm, cm = pl.pallas_call(
        _mid_k,
        grid=(_B,),
        in_specs=[
            pl.BlockSpec((1, _L, _DI), lambda b: (b, 0, 0)),
            pl.BlockSpec((_DC, _DI), lambda b: (0, 0)),
            pl.BlockSpec((1, _DI), lambda b: (0, 0)),
            pl.BlockSpec((_DR + 2 * _DS, _DI), lambda b: (0, 0)),
            pl.BlockSpec((_DI, _DR), lambda b: (0, 0)),
            pl.BlockSpec((1, _DI), lambda b: (0, 0)),
        ],
        out_specs=[
            pl.BlockSpec((1, _L, _DI), lambda b: (b, 0, 0)),
            pl.BlockSpec((1, _L, _DI), lambda b: (b, 0, 0)),
            pl.BlockSpec((1, _L, _DS), lambda b: (b, 0, 0)),
            pl.BlockSpec((1, _L, _DS), lambda b: (b, 0, 0)),
        ],
        out_shape=[
            jax.ShapeDtypeStruct((_B, _L, _DI), f32),
            jax.ShapeDtypeStruct((_B, _L, _DI), f32),
            jax.ShapeDtypeStruct((_B, _L, _DS), f32),
            jax.ShapeDtypeStruct((_B, _L, _DS), f32),
        ],
        compiler_params=_cp(("parallel",)),
    )(xz, cw.T, cb.reshape(1, _DI), wx, wdt, bdt.reshape(1, _DI))

    bct = jnp.concatenate(
        [bm.reshape(_B, _NCH, _TCH, _DS).transpose(0, 1, 3, 2),
         cm.reshape(_B, _NCH, _TCH, _DS).transpose(0, 1, 3, 2)], axis=2)

    return pl.pallas_call(
        _scan_out_k,
        grid=(_B, _NJ),
        in_specs=[
            pl.BlockSpec((1, _L, _DBLK), lambda b, j: (b, 0, j)),
            pl.BlockSpec((1, _L, _DBLK), lambda b, j: (b, 0, j)),
            pl.BlockSpec((1, _L, _DBLK), lambda b, j: (b, 0, _NJ + j)),
            pl.BlockSpec((1, _NCH, 2 * _DS, _TCH), lambda b, j: (b, 0, 0, 0)),
            pl.BlockSpec((_DS, _DBLK), lambda b, j: (0, j)),
            pl.BlockSpec((1, _DBLK), lambda b, j: (0, j)),
            pl.BlockSpec((_DM, _DBLK), lambda b, j: (0, j)),
        ],
        out_specs=pl.BlockSpec((1, _L, _DM), lambda b, j: (b, 0, 0)),
        out_shape=jax.ShapeDtypeStruct((_B, _L, _DM), f32),
        scratch_shapes=[pltpu.VMEM((_L, _DBLK), f32)],
        compiler_params=_cp(("parallel", "arbitrary")),
    )(dlt, u, xz, bct, alog.T, dvec.reshape(1, _DI), wo)


def kernel(x, in_proj_w, conv_w, conv_b, x_proj_w, dt_proj_w, dt_proj_b,
           A_log, D, out_proj_w):
    for i in range(4):
        x = _layer(x, in_proj_w[i], conv_w[i], conv_b[i], x_proj_w[i],
                   dt_proj_w[i], dt_proj_b[i], A_log[i], D[i], out_proj_w[i])
    return x
